# baseline (speedup 1.0000x reference)
import functools

import jax
import jax.numpy as jnp
from jax import lax
from jax.experimental import pallas as pl
from jax.experimental.pallas import tpu as pltpu
from jax.experimental.pallas import tpu_sc as plsc

VOCAB = 1000
VPAD = 1024
N_TOK = 1024 * 50

NC = 2
NS = 16
NW = NC * NS
L = 16

ROWS_PER_W = N_TOK // NW

_MESH = plsc.VectorSubcoreMesh(core_axis_name="c", subcore_axis_name="s")


HCH = 64
NHCH = ROWS_PER_W // HCH


def _hist_body(idx_hbm, tgt_hbm, zeros_hbm, hp_hbm,
               idx_v, tgt_v, flat_v, ones_v, sh):
    cid = lax.axis_index("c")
    sid = lax.axis_index("s")
    wid = sid * NC + cid
    base = wid * ROWS_PER_W

    @pl.when(sid == 0)
    def _():
        pltpu.sync_copy(zeros_hbm, sh)

    plsc.subcore_barrier()

    pltpu.sync_copy(idx_hbm.at[pl.ds(base, ROWS_PER_W)], idx_v)
    pltpu.sync_copy(tgt_hbm.at[pl.ds(base, ROWS_PER_W)], tgt_v)
    for k in range(HCH // L):
        ones_v[pl.ds(k * L, L)] = jnp.ones((L,), jnp.float32)

    def chunk(c, carry):
        for k in range(HCH // L):
            o = c * HCH + k * L
            i16 = idx_v[pl.ds(o, L)]
            t16 = tgt_v[pl.ds(o, L)]
            flat_v[pl.ds(k * L, L)] = i16 * VOCAB + t16
        pltpu.sync_copy(ones_v, sh.at[flat_v], add=True)
        return carry

    lax.fori_loop(0, NHCH, chunk, 0)
    plsc.subcore_barrier()

    @pl.when(sid == 0)
    def _():
        pltpu.sync_copy(sh, hp_hbm.at[cid])


_hist_call = functools.partial(
    pl.kernel,
    out_type=jax.ShapeDtypeStruct((NC, VOCAB * VOCAB), jnp.float32),
    mesh=_MESH,
    compiler_params=pltpu.CompilerParams(
        use_tc_tiling_on_sc=False, needs_layout_passes=False),
    scratch_types=[
        pltpu.VMEM((ROWS_PER_W,), jnp.int32),
        pltpu.VMEM((ROWS_PER_W,), jnp.int32),
        pltpu.VMEM((HCH,), jnp.int32),
        pltpu.VMEM((HCH,), jnp.float32),
        pltpu.VMEM_SHARED((VOCAB * VOCAB,), jnp.float32),
    ],
)(_hist_body)


CH = 40

VMAIN = 896
VREM = VPAD - VMAIN


def _make_gather(n_part):
    rows_per_w = n_part // NW
    nch = rows_per_w // CH
    assert rows_per_w % CH == 0 and nch % 2 == 0

    def _gather_body(emb_hbm, idx_hbm, outm_hbm, outr_hbm,
                     idx_v, rows0, rows1, gs0, gs1, ws0, ws1):
        wid = lax.axis_index("s") * NC + lax.axis_index("c")
        base = wid * rows_per_w
        pltpu.sync_copy(idx_hbm.at[pl.ds(base, rows_per_w)], idx_v)
        rows = (rows0, rows1)
        gs = (gs0, gs1)
        ws = (ws0, ws1)

        def gather(c, b):
            return pltpu.make_async_copy(
                emb_hbm.at[idx_v.at[pl.ds(c * CH, CH)]], rows[b], gs[b])

        def write_m(c, b):
            return pltpu.make_async_copy(
                rows[b].at[pl.ds(0, CH), pl.ds(0, VMAIN)],
                outm_hbm.at[pl.ds(base + c * CH, CH), pl.ds(0, VMAIN)],
                ws[b])

        def write_r(c, b):
            return pltpu.make_async_copy(
                rows[b].at[pl.ds(0, CH), pl.ds(VMAIN, VREM)],
                outr_hbm.at[pl.ds(base + c * CH, CH)], ws[b])

        gather(0, 0).start()
        gather(1, 1).start()

        def pair(i, carry):
            for b in range(2):
                c = 2 * i + b
                gather(c, b).wait()
                write_m(c, b).start()
                write_r(c, b).start()
                write_m(c, b).wait()
                write_r(c, b).wait()
                gather(c + 2, b).start()
            return carry

        lax.fori_loop(0, nch // 2 - 1, pair, 0)
        for b in range(2):
            c = nch - 2 + b
            gather(c, b).wait()
            write_m(c, b).start()
            write_r(c, b).start()
            write_m(c, b).wait()
            write_r(c, b).wait()

    return functools.partial(
        pl.kernel,
        out_type=[
            jax.ShapeDtypeStruct((n_part, VOCAB), jnp.float32),
            jax.ShapeDtypeStruct((n_part, VREM), jnp.float32),
        ],
        mesh=_MESH,
        scratch_types=[
            pltpu.VMEM((rows_per_w,), jnp.int32),
            pltpu.VMEM((CH, VPAD), jnp.float32),
            pltpu.VMEM((CH, VPAD), jnp.float32),
            pltpu.SemaphoreType.DMA,
            pltpu.SemaphoreType.DMA,
            pltpu.SemaphoreType.DMA,
            pltpu.SemaphoreType.DMA,
        ],
    )(_gather_body)


NSPLIT = 2
N_PART = N_TOK // NSPLIT
_gather_part = _make_gather(N_PART)


def _loss_body(emb_ref, h_ref, loss_ref):
    x = emb_ref[...]
    m = jnp.max(x, axis=1, keepdims=True)
    s = jnp.sum(jnp.exp(x - m), axis=1, keepdims=True)
    lse = jnp.log(s) + m
    h = h_ref[0] + h_ref[1]
    counts = jnp.sum(h, axis=1, keepdims=True)
    total = jnp.sum(counts * lse) - jnp.sum(h * x)
    loss_ref[...] = jnp.reshape(total / jnp.float32(N_TOK), (1, 1))


_loss_call = pl.pallas_call(
    _loss_body,
    out_shape=jax.ShapeDtypeStruct((1, 1), jnp.float32),
)


def kernel(idx, targets, embedding):
    idxf = idx.reshape(-1).astype(jnp.int32)
    tgtf = targets.reshape(-1).astype(jnp.int32)
    embp = jnp.pad(embedding, ((0, 0), (0, VPAD - VOCAB)))
    hp = _hist_call(idxf, tgtf, jnp.zeros((VOCAB * VOCAB,), jnp.float32))
    parts = []
    for s in range(NSPLIT):
        outm, outr = _gather_part(embp, idxf[s * N_PART:(s + 1) * N_PART])
        parts.append(lax.dynamic_update_slice(
            outm, outr[:, :VOCAB - VMAIN], (0, VMAIN)))
    logits = jnp.concatenate(parts, axis=0)
    loss = _loss_call(embedding, hp.reshape(NC, VOCAB, VOCAB))[0, 0]
    return (logits, loss)


# speedup vs baseline: 1.3025x; 1.3025x over previous
import functools

import jax
import jax.numpy as jnp
from jax import lax
from jax.experimental import pallas as pl
from jax.experimental.pallas import tpu as pltpu
from jax.experimental.pallas import tpu_sc as plsc

VOCAB = 1000
VPAD = 1024
N_TOK = 1024 * 50

NC = 2
NS = 16
NW = NC * NS
L = 16

ROWS_PER_W = N_TOK // NW

_MESH = plsc.VectorSubcoreMesh(core_axis_name="c", subcore_axis_name="s")


HCH = 64
NHCH = ROWS_PER_W // HCH


def _hist_body(idx_hbm, tgt_hbm, zeros_hbm, hp_hbm,
               idx_v, tgt_v, flat_v, ones_v, sh):
    cid = lax.axis_index("c")
    sid = lax.axis_index("s")
    wid = sid * NC + cid
    base = wid * ROWS_PER_W

    @pl.when(sid == 0)
    def _():
        pltpu.sync_copy(zeros_hbm, sh)

    plsc.subcore_barrier()

    pltpu.sync_copy(idx_hbm.at[pl.ds(base, ROWS_PER_W)], idx_v)
    pltpu.sync_copy(tgt_hbm.at[pl.ds(base, ROWS_PER_W)], tgt_v)
    for k in range(HCH // L):
        ones_v[pl.ds(k * L, L)] = jnp.ones((L,), jnp.float32)

    def chunk(c, carry):
        for k in range(HCH // L):
            o = c * HCH + k * L
            i16 = idx_v[pl.ds(o, L)]
            t16 = tgt_v[pl.ds(o, L)]
            flat_v[pl.ds(k * L, L)] = i16 * VOCAB + t16
        pltpu.sync_copy(ones_v, sh.at[flat_v], add=True)
        return carry

    lax.fori_loop(0, NHCH, chunk, 0)
    plsc.subcore_barrier()

    @pl.when(sid == 0)
    def _():
        pltpu.sync_copy(sh, hp_hbm.at[cid])


_hist_call = functools.partial(
    pl.kernel,
    out_type=jax.ShapeDtypeStruct((NC, VOCAB * VOCAB), jnp.float32),
    mesh=_MESH,
    compiler_params=pltpu.CompilerParams(
        use_tc_tiling_on_sc=False, needs_layout_passes=False),
    scratch_types=[
        pltpu.VMEM((ROWS_PER_W,), jnp.int32),
        pltpu.VMEM((ROWS_PER_W,), jnp.int32),
        pltpu.VMEM((HCH,), jnp.int32),
        pltpu.VMEM((HCH,), jnp.float32),
        pltpu.VMEM_SHARED((VOCAB * VOCAB,), jnp.float32),
    ],
)(_hist_body)


CH = 40

VMAIN = 896
VREM = VPAD - VMAIN


def _make_gather(n_part):
    rows_per_w = n_part // NW
    nch = rows_per_w // CH
    assert rows_per_w % CH == 0 and nch % 2 == 0

    def _gather_body(emb_hbm, idx_hbm, outm_hbm, outr_hbm,
                     idx_v, rows0, rows1, gs0, gs1, ws0, ws1):
        wid = lax.axis_index("s") * NC + lax.axis_index("c")
        base = wid * rows_per_w
        pltpu.sync_copy(idx_hbm.at[pl.ds(base, rows_per_w)], idx_v)
        rows = (rows0, rows1)
        gs = (gs0, gs1)
        ws = (ws0, ws1)

        def gather(c, b):
            return pltpu.make_async_copy(
                emb_hbm.at[idx_v.at[pl.ds(c * CH, CH)]], rows[b], gs[b])

        def write_m(c, b):
            return pltpu.make_async_copy(
                rows[b].at[pl.ds(0, CH), pl.ds(0, VMAIN)],
                outm_hbm.at[pl.ds(base + c * CH, CH), pl.ds(0, VMAIN)],
                ws[b])

        def write_r(c, b):
            return pltpu.make_async_copy(
                rows[b].at[pl.ds(0, CH), pl.ds(VMAIN, VREM)],
                outr_hbm.at[pl.ds(base + c * CH, CH)], ws[b])

        gather(0, 0).start()
        gather(1, 1).start()

        def pair(i, carry):
            for b in range(2):
                c = 2 * i + b
                gather(c, b).wait()
                write_m(c, b).start()
                write_r(c, b).start()
                write_m(c, b).wait()
                write_r(c, b).wait()
                gather(c + 2, b).start()
            return carry

        lax.fori_loop(0, nch // 2 - 1, pair, 0)
        for b in range(2):
            c = nch - 2 + b
            gather(c, b).wait()
            write_m(c, b).start()
            write_r(c, b).start()
            write_m(c, b).wait()
            write_r(c, b).wait()

    return functools.partial(
        pl.kernel,
        out_type=[
            jax.ShapeDtypeStruct((n_part, VOCAB), jnp.float32),
            jax.ShapeDtypeStruct((n_part, VREM), jnp.float32),
        ],
        mesh=_MESH,
        scratch_types=[
            pltpu.VMEM((rows_per_w,), jnp.int32),
            pltpu.VMEM((CH, VPAD), jnp.float32),
            pltpu.VMEM((CH, VPAD), jnp.float32),
            pltpu.SemaphoreType.DMA,
            pltpu.SemaphoreType.DMA,
            pltpu.SemaphoreType.DMA,
            pltpu.SemaphoreType.DMA,
        ],
    )(_gather_body)


_gather_part = _make_gather(N_TOK)


def _loss_body(emb_ref, h_ref, loss_ref):
    x = emb_ref[...]
    m = jnp.max(x, axis=1, keepdims=True)
    s = jnp.sum(jnp.exp(x - m), axis=1, keepdims=True)
    lse = jnp.log(s) + m
    h = h_ref[0] + h_ref[1]
    counts = jnp.sum(h, axis=1, keepdims=True)
    total = jnp.sum(counts * lse) - jnp.sum(h * x)
    loss_ref[...] = jnp.reshape(total / jnp.float32(N_TOK), (1, 1))


_loss_call = pl.pallas_call(
    _loss_body,
    out_shape=jax.ShapeDtypeStruct((1, 1), jnp.float32),
)


def kernel(idx, targets, embedding):
    idxf = idx.reshape(-1).astype(jnp.int32)
    tgtf = targets.reshape(-1).astype(jnp.int32)
    embp = jnp.pad(embedding, ((0, 0), (0, VPAD - VOCAB)))
    hp = _hist_call(idxf, tgtf, jnp.zeros((VOCAB * VOCAB,), jnp.float32))
    idx_dep = idxf + (hp[0, 0] * 0.0).astype(jnp.int32)
    outm, outr = _gather_part(embp, idx_dep)
    logits = lax.dynamic_update_slice(outm, outr[:, :VOCAB - VMAIN], (0, VMAIN))
    loss = _loss_call(embedding, hp.reshape(NC, VOCAB, VOCAB))[0, 0]
    return (logits, loss)


# speedup vs baseline: 1.3839x; 1.0624x over previous
import functools

import jax
import jax.numpy as jnp
from jax import lax
from jax.experimental import pallas as pl
from jax.experimental.pallas import tpu as pltpu
from jax.experimental.pallas import tpu_sc as plsc

VOCAB = 1000
VPAD = 1024
N_TOK = 1024 * 50

NC = 2
NS = 16
NW = NC * NS
L = 16

ROWS_PER_W = N_TOK // NW

_MESH = plsc.VectorSubcoreMesh(core_axis_name="c", subcore_axis_name="s")


HCH = 64
NHCH = ROWS_PER_W // HCH


def _hist_body(idx_hbm, tgt_hbm, zeros_hbm, hp_hbm,
               idx_v, tgt_v, flat_v, ones_v, sh):
    cid = lax.axis_index("c")
    sid = lax.axis_index("s")
    wid = sid * NC + cid
    base = wid * ROWS_PER_W

    @pl.when(sid == 0)
    def _():
        pltpu.sync_copy(zeros_hbm, sh)

    plsc.subcore_barrier()

    pltpu.sync_copy(idx_hbm.at[pl.ds(base, ROWS_PER_W)], idx_v)
    pltpu.sync_copy(tgt_hbm.at[pl.ds(base, ROWS_PER_W)], tgt_v)
    for k in range(HCH // L):
        ones_v[pl.ds(k * L, L)] = jnp.ones((L,), jnp.float32)

    def chunk(c, carry):
        for k in range(HCH // L):
            o = c * HCH + k * L
            i16 = idx_v[pl.ds(o, L)]
            t16 = tgt_v[pl.ds(o, L)]
            flat_v[pl.ds(k * L, L)] = i16 * VOCAB + t16
        pltpu.sync_copy(ones_v, sh.at[flat_v], add=True)
        return carry

    lax.fori_loop(0, NHCH, chunk, 0)
    plsc.subcore_barrier()

    @pl.when(sid == 0)
    def _():
        pltpu.sync_copy(sh, hp_hbm.at[cid])


_hist_call = functools.partial(
    pl.kernel,
    out_type=jax.ShapeDtypeStruct((NC, VOCAB * VOCAB), jnp.float32),
    mesh=_MESH,
    compiler_params=pltpu.CompilerParams(
        use_tc_tiling_on_sc=False, needs_layout_passes=False),
    scratch_types=[
        pltpu.VMEM((ROWS_PER_W,), jnp.int32),
        pltpu.VMEM((ROWS_PER_W,), jnp.int32),
        pltpu.VMEM((HCH,), jnp.int32),
        pltpu.VMEM((HCH,), jnp.float32),
        pltpu.VMEM_SHARED((VOCAB * VOCAB,), jnp.float32),
    ],
)(_hist_body)


CH = 40

VMAIN = 896
VREM = VPAD - VMAIN


def _make_gather(n_part):
    rows_per_w = n_part // NW
    nch = rows_per_w // CH
    assert rows_per_w % CH == 0 and nch % 2 == 0

    def _gather_body(emb_hbm, idx_hbm, outm_hbm, outr_hbm,
                     idx_v, rows0, rows1, gs0, gs1, ws0, ws1):
        wid = lax.axis_index("s") * NC + lax.axis_index("c")
        base = wid * rows_per_w
        pltpu.sync_copy(idx_hbm.at[pl.ds(base, rows_per_w)], idx_v)
        rows = (rows0, rows1)
        gs = (gs0, gs1)
        ws = (ws0, ws1)

        def gather(c, b):
            return pltpu.make_async_copy(
                emb_hbm.at[idx_v.at[pl.ds(c * CH, CH)]], rows[b], gs[b])

        def write_m(c, b):
            return pltpu.make_async_copy(
                rows[b].at[pl.ds(0, CH), pl.ds(0, VMAIN)],
                outm_hbm.at[pl.ds(base + c * CH, CH), pl.ds(0, VMAIN)],
                ws[b])

        def write_r(c, b):
            return pltpu.make_async_copy(
                rows[b].at[pl.ds(0, CH), pl.ds(VMAIN, VREM)],
                outr_hbm.at[pl.ds(base + c * CH, CH)], ws[b])

        gather(0, 0).start()
        gather(1, 1).start()

        def pair(i, carry):
            for b in range(2):
                c = 2 * i + b
                gather(c, b).wait()
                write_m(c, b).start()
                write_r(c, b).start()
                write_m(c, b).wait()
                write_r(c, b).wait()
                gather(c + 2, b).start()
            return carry

        lax.fori_loop(0, nch // 2 - 1, pair, 0)
        for b in range(2):
            c = nch - 2 + b
            gather(c, b).wait()
            write_m(c, b).start()
            write_r(c, b).start()
            write_m(c, b).wait()
            write_r(c, b).wait()

    return functools.partial(
        pl.kernel,
        out_type=[
            jax.ShapeDtypeStruct((n_part, VOCAB), jnp.float32),
            jax.ShapeDtypeStruct((n_part, VREM), jnp.float32),
        ],
        mesh=_MESH,
        scratch_types=[
            pltpu.VMEM((rows_per_w,), jnp.int32),
            pltpu.VMEM((CH, VPAD), jnp.float32),
            pltpu.VMEM((CH, VPAD), jnp.float32),
            pltpu.SemaphoreType.DMA,
            pltpu.SemaphoreType.DMA,
            pltpu.SemaphoreType.DMA,
            pltpu.SemaphoreType.DMA,
        ],
    )(_gather_body)


_gather_part = _make_gather(N_TOK)


def _loss_body(emb_ref, h_ref, loss_ref):
    x = emb_ref[...]
    m = jnp.max(x, axis=1, keepdims=True)
    s = jnp.sum(jnp.exp(x - m), axis=1, keepdims=True)
    lse = jnp.log(s) + m
    h = h_ref[0] + h_ref[1]
    counts = jnp.sum(h, axis=1, keepdims=True)
    total = jnp.sum(counts * lse) - jnp.sum(h * x)
    loss_ref[...] = jnp.reshape(total / jnp.float32(N_TOK), (1, 1))


_loss_call = pl.pallas_call(
    _loss_body,
    out_shape=jax.ShapeDtypeStruct((1, 1), jnp.float32),
)


def kernel(idx, targets, embedding):
    idxf = idx.reshape(-1).astype(jnp.int32)
    tgtf = targets.reshape(-1).astype(jnp.int32)
    embp = jnp.pad(embedding, ((0, 0), (0, VPAD - VOCAB)))
    hp = _hist_call(idxf, tgtf, jnp.zeros((VOCAB * VOCAB,), jnp.float32))
    outm, outr = _gather_part(embp, idxf)
    logits = lax.dynamic_update_slice(outm, outr[:, :VOCAB - VMAIN], (0, VMAIN))
    loss = _loss_call(embedding, hp.reshape(NC, VOCAB, VOCAB))[0, 0]
    return (logits, loss)


# speedup vs baseline: 1.3949x; 1.0080x over previous
import functools

import jax
import jax.numpy as jnp
from jax import lax
from jax.experimental import pallas as pl
from jax.experimental.pallas import tpu as pltpu
from jax.experimental.pallas import tpu_sc as plsc

VOCAB = 1000
VPAD = 1024
N_TOK = 1024 * 50

NC = 2
NS = 16
NW = NC * NS
L = 16

ROWS_PER_W = N_TOK // NW

_MESH = plsc.VectorSubcoreMesh(core_axis_name="c", subcore_axis_name="s")


HCH = 64
NHCH = ROWS_PER_W // HCH


def _hist_body(idx_hbm, tgt_hbm, zeros_hbm, hp_hbm,
               idx_v, tgt_v, flat_v, ones_v, sh):
    cid = lax.axis_index("c")
    sid = lax.axis_index("s")
    wid = sid * NC + cid
    base = wid * ROWS_PER_W

    @pl.when(sid == 0)
    def _():
        pltpu.sync_copy(zeros_hbm, sh)

    plsc.subcore_barrier()

    pltpu.sync_copy(idx_hbm.at[pl.ds(base, ROWS_PER_W)], idx_v)
    pltpu.sync_copy(tgt_hbm.at[pl.ds(base, ROWS_PER_W)], tgt_v)
    for k in range(HCH // L):
        ones_v[pl.ds(k * L, L)] = jnp.ones((L,), jnp.float32)

    def chunk(c, carry):
        for k in range(HCH // L):
            o = c * HCH + k * L
            i16 = idx_v[pl.ds(o, L)]
            t16 = tgt_v[pl.ds(o, L)]
            flat_v[pl.ds(k * L, L)] = i16 * VOCAB + t16
        pltpu.sync_copy(ones_v, sh.at[flat_v], add=True)
        return carry

    lax.fori_loop(0, NHCH, chunk, 0)
    plsc.subcore_barrier()

    @pl.when(sid == 0)
    def _():
        pltpu.sync_copy(sh, hp_hbm.at[cid])


_hist_call = functools.partial(
    pl.kernel,
    out_type=jax.ShapeDtypeStruct((NC, VOCAB * VOCAB), jnp.float32),
    mesh=_MESH,
    compiler_params=pltpu.CompilerParams(
        use_tc_tiling_on_sc=False, needs_layout_passes=False),
    scratch_types=[
        pltpu.VMEM((ROWS_PER_W,), jnp.int32),
        pltpu.VMEM((ROWS_PER_W,), jnp.int32),
        pltpu.VMEM((HCH,), jnp.int32),
        pltpu.VMEM((HCH,), jnp.float32),
        pltpu.VMEM_SHARED((VOCAB * VOCAB,), jnp.float32),
    ],
)(_hist_body)


CH = 32

VMAIN = 896
VREM = VPAD - VMAIN


def _make_gather(n_part):
    rows_per_w = n_part // NW
    nch = rows_per_w // CH
    assert rows_per_w % CH == 0 and nch % 2 == 0

    def _gather_body(emb_hbm, idx_hbm, outm_hbm, outr_hbm,
                     idx_v, rows0, rows1, gs0, gs1, ws0, ws1):
        wid = lax.axis_index("s") * NC + lax.axis_index("c")
        base = wid * rows_per_w
        pltpu.sync_copy(idx_hbm.at[pl.ds(base, rows_per_w)], idx_v)
        rows = (rows0, rows1)
        gs = (gs0, gs1)
        ws = (ws0, ws1)

        def gather(c, b):
            return pltpu.make_async_copy(
                emb_hbm.at[idx_v.at[pl.ds(c * CH, CH)]], rows[b], gs[b])

        def write_m(c, b):
            return pltpu.make_async_copy(
                rows[b].at[pl.ds(0, CH), pl.ds(0, VMAIN)],
                outm_hbm.at[pl.ds(base + c * CH, CH), pl.ds(0, VMAIN)],
                ws[b])

        def write_r(c, b):
            return pltpu.make_async_copy(
                rows[b].at[pl.ds(0, CH), pl.ds(VMAIN, VREM)],
                outr_hbm.at[pl.ds(base + c * CH, CH)], ws[b])

        gather(0, 0).start()
        gather(1, 1).start()

        def pair(i, carry):
            for b in range(2):
                c = 2 * i + b
                gather(c, b).wait()
                write_m(c, b).start()
                write_r(c, b).start()
                write_m(c, b).wait()
                write_r(c, b).wait()
                gather(c + 2, b).start()
            return carry

        lax.fori_loop(0, nch // 2 - 1, pair, 0)
        for b in range(2):
            c = nch - 2 + b
            gather(c, b).wait()
            write_m(c, b).start()
            write_r(c, b).start()
            write_m(c, b).wait()
            write_r(c, b).wait()

    return functools.partial(
        pl.kernel,
        out_type=[
            jax.ShapeDtypeStruct((n_part, VOCAB), jnp.float32),
            jax.ShapeDtypeStruct((n_part, VREM), jnp.float32),
        ],
        mesh=_MESH,
        scratch_types=[
            pltpu.VMEM((rows_per_w,), jnp.int32),
            pltpu.VMEM((CH, VPAD), jnp.float32),
            pltpu.VMEM((CH, VPAD), jnp.float32),
            pltpu.SemaphoreType.DMA,
            pltpu.SemaphoreType.DMA,
            pltpu.SemaphoreType.DMA,
            pltpu.SemaphoreType.DMA,
        ],
    )(_gather_body)


_gather_part = _make_gather(N_TOK)


def _loss_body(emb_ref, h_ref, loss_ref):
    x = emb_ref[...]
    m = jnp.max(x, axis=1, keepdims=True)
    s = jnp.sum(jnp.exp(x - m), axis=1, keepdims=True)
    lse = jnp.log(s) + m
    h = h_ref[0] + h_ref[1]
    counts = jnp.sum(h, axis=1, keepdims=True)
    total = jnp.sum(counts * lse) - jnp.sum(h * x)
    loss_ref[...] = jnp.reshape(total / jnp.float32(N_TOK), (1, 1))


_loss_call = pl.pallas_call(
    _loss_body,
    out_shape=jax.ShapeDtypeStruct((1, 1), jnp.float32),
)


def kernel(idx, targets, embedding):
    idxf = idx.reshape(-1).astype(jnp.int32)
    tgtf = targets.reshape(-1).astype(jnp.int32)
    embp = jnp.pad(embedding, ((0, 0), (0, VPAD - VOCAB)))
    hp = _hist_call(idxf, tgtf, jnp.zeros((VOCAB * VOCAB,), jnp.float32))
    outm, outr = _gather_part(embp, idxf)
    logits = lax.dynamic_update_slice(outm, outr[:, :VOCAB - VMAIN], (0, VMAIN))
    loss = _loss_call(embedding, hp.reshape(NC, VOCAB, VOCAB))[0, 0]
    return (logits, loss)
